# baseline (speedup 1.0000x reference)
import jax
import jax.numpy as jnp
from jax import lax
from jax.experimental import pallas as pl
from jax.experimental.pallas import tpu as pltpu
from jax.experimental.pallas import tpu_sc as plsc

_B, _K, _N, _H, _W = 2, 128, 4096, 512, 512
_NCLS = 10
_NC, _NS = 2, 16
_NW = _NC * _NS
_NLOC = (_B * _N) // _NW
_NGRP = _NLOC // 16
_GPI = 4
_NINST = _NGRP // _GPI


def _rsqrt(s):
    i = plsc.bitcast(s, jnp.int32)
    i = jnp.int32(0x5F3759DF) - (i >> 1)
    y = plsc.bitcast(i, jnp.float32)
    for _ in range(3):
        y = y * (jnp.float32(1.5) - jnp.float32(0.5) * s * y * y)
    return y


def _splat(v):
    return jnp.full((16,), v, jnp.int32)


def _vote_fusion_body(boxes_hbm, s2_hbm, s3_hbm, img_hbm, fu_hbm, out_hbm,
                      boxes_v, midx_v, midy_v, wk2_v, hk2_v, s2_v, s3_v,
                      fu_v, idx_refs, txt_v, bidx_v, macc_v, out_v, sem):
    wid = lax.axis_index("s") * _NC + lax.axis_index("c")
    b = wid // _NS
    base = (wid % _NS) * _NLOC

    pltpu.sync_copy(boxes_hbm.at[pl.ds(b * 6 * _K, 6 * _K)], boxes_v)
    for rr in range(2):
        pltpu.sync_copy(s2_hbm.at[pl.ds((b * 2 + rr) * _N + base, _NLOC)],
                        s2_v.at[pl.ds(rr * _NLOC, _NLOC)])
    for rr in range(3):
        pltpu.sync_copy(s3_hbm.at[pl.ds((b * 3 + rr) * _N + base, _NLOC)],
                        s3_v.at[pl.ds(rr * _NLOC, _NLOC)])
    pltpu.sync_copy(fu_hbm.at[pl.ds(b * 16, 16)], fu_v)

    for i in range(_K // 16):
        sl = pl.ds(i * 16, 16)
        l = boxes_v[pl.ds(0 * _K + i * 16, 16)]
        t = boxes_v[pl.ds(1 * _K + i * 16, 16)]
        r = boxes_v[pl.ds(2 * _K + i * 16, 16)]
        bo = boxes_v[pl.ds(3 * _K + i * 16, 16)]
        midx_v[sl] = (l + r) * 0.5
        midy_v[sl] = (t + bo) * 0.5
        wk2 = (r - l) * 0.5
        hk2 = (bo - t) * 0.5
        wk2_v[sl] = wk2 * wk2
        hk2_v[sl] = hk2 * hk2

    iota = jnp.arange(16, dtype=jnp.int32)
    for g in range(_NGRP):
        xi = s2_v[pl.ds(g * 16, 16)].astype(jnp.int32)
        yi = s2_v[pl.ds(_NLOC + g * 16, 16)].astype(jnp.int32)
        pix = jnp.minimum(jnp.maximum(yi * _W + xi, 0), _H * _W - 1)
        for c in range(3):
            r = c * 2 + g // 8
            idx_refs[r][pl.ds((g % 8) * 16, 16)] = pix + (b * 3 + c) * (_H * _W)
    copies = []
    for c in range(3):
        for j in range(2):
            copies.append(pltpu.async_copy(
                img_hbm.at[idx_refs[c * 2 + j]],
                txt_v.at[pl.ds((c * 2 + j) * 128, 128)], sem))

    for inst in range(_NINST):
        sxs = [s2_v[pl.ds(inst * 64 + j * 16, 16)] for j in range(_GPI)]
        sys_ = [s2_v[pl.ds(_NLOC + inst * 64 + j * 16, 16)]
                for j in range(_GPI)]
        inf = jnp.full((16,), jnp.inf, jnp.float32)
        zero = jnp.zeros((16,), jnp.int32)

        def body(k, carry, sxs=sxs, sys_=sys_):
            best, bidx, macc = carry
            kk = jnp.full((16,), k, jnp.int32)
            mx = plsc.load_gather(midx_v, [kk])
            my = plsc.load_gather(midy_v, [kk])
            wk2s = plsc.load_gather(wk2_v, [kk])
            hk2s = plsc.load_gather(hk2_v, [kk])
            nb, nbi, nm = [], [], []
            for j in range(_GPI):
                du = mx - sxs[j]
                dv = my - sys_[j]
                du2 = du * du
                dv2 = dv * dv
                d2 = du2 + dv2
                upd = d2 < best[j]
                nbi.append(jnp.where(upd, kk, bidx[j]))
                nb.append(jnp.minimum(d2, best[j]))
                m = jnp.maximum(du2 - wk2s, dv2 - hk2s)
                nm.append(jnp.minimum(m, macc[j]))
            return tuple(nb), tuple(nbi), tuple(nm)

        init = ((inf,) * _GPI, (zero,) * _GPI, (inf,) * _GPI)
        _, bidx, macc = lax.fori_loop(0, _K, body, init, unroll=4)
        for j in range(_GPI):
            sl = pl.ds(inst * 64 + j * 16, 16)
            bidx_v[sl] = bidx[j]
            macc_v[sl] = macc[j]

    for cp in copies:
        cp.wait()

    fu = fu_v[...]
    for g in range(_NGRP):
        sx = s2_v[pl.ds(g * 16, 16)]
        sy = s2_v[pl.ds(_NLOC + g * 16, 16)]
        x3 = s3_v[pl.ds(g * 16, 16)]
        y3 = s3_v[pl.ds(_NLOC + g * 16, 16)]
        z3 = s3_v[pl.ds(2 * _NLOC + g * 16, 16)]
        bidx = bidx_v[pl.ds(g * 16, 16)]
        valid = macc_v[pl.ds(g * 16, 16)] < 0.0
        mxa = plsc.load_gather(midx_v, [bidx])
        mya = plsc.load_gather(midy_v, [bidx])
        confa = plsc.load_gather(boxes_v, [bidx + 4 * _K])
        clsa = plsc.load_gather(boxes_v, [bidx + 5 * _K])
        du = mxa - sx
        dv = mya - sy
        zdf = z3 / fu
        g2 = du * zdf + x3
        g3 = dv * zdf + y3
        inv = _rsqrt(g2 * g2 + g3 * g3 + z3 * z3)
        rows18 = (iota + g * 16) * 18
        zf = jnp.zeros((16,), jnp.float32)
        for c in range(_NCLS):
            val = jnp.where(valid & (clsa == float(c)), confa, zf)
            plsc.store_scatter(out_v, [rows18 + c], val)
        for c in range(3):
            val = jnp.where(valid, txt_v[pl.ds(c * _NLOC + g * 16, 16)], zf)
            plsc.store_scatter(out_v, [rows18 + (10 + c)], val)
        geo = (g2, g3, g2 * inv, g3 * inv, z3 * inv)
        for c in range(5):
            val = jnp.where(valid, geo[c], zf)
            plsc.store_scatter(out_v, [rows18 + (13 + c)], val)

    pltpu.sync_copy(out_v, out_hbm.at[pl.ds((b * _N + base) * 18, _NLOC * 18)])


def _body_wrapper(boxes_hbm, s2_hbm, s3_hbm, img_hbm, fu_hbm, out_hbm,
                  boxes_v, midx_v, midy_v, wk2_v, hk2_v, s2_v, s3_v, fu_v,
                  i0, i1, i2, i3, i4, i5, txt_v, bidx_v, macc_v, out_v, sem):
    _vote_fusion_body(boxes_hbm, s2_hbm, s3_hbm, img_hbm, fu_hbm, out_hbm,
                      boxes_v, midx_v, midy_v, wk2_v, hk2_v, s2_v, s3_v,
                      fu_v, [i0, i1, i2, i3, i4, i5], txt_v, bidx_v, macc_v,
                      out_v, sem)


@jax.jit
def _vote_fusion(boxes_t, s2_t, s3_t, img_flat, fu_tile):
    run = pl.kernel(
        _body_wrapper,
        out_type=jax.ShapeDtypeStruct((_B * _N * 18,), jnp.float32),
        mesh=plsc.VectorSubcoreMesh(
            core_axis_name="c", subcore_axis_name="s",
            num_cores=_NC, num_subcores=_NS),
        compiler_params=pltpu.CompilerParams(needs_layout_passes=False),
        scratch_types=[
            pltpu.VMEM((6 * _K,), jnp.float32),
            pltpu.VMEM((_K,), jnp.float32),
            pltpu.VMEM((_K,), jnp.float32),
            pltpu.VMEM((_K,), jnp.float32),
            pltpu.VMEM((_K,), jnp.float32),
            pltpu.VMEM((2 * _NLOC,), jnp.float32),
            pltpu.VMEM((3 * _NLOC,), jnp.float32),
            pltpu.VMEM((16,), jnp.float32),
            pltpu.VMEM((128,), jnp.int32),
            pltpu.VMEM((128,), jnp.int32),
            pltpu.VMEM((128,), jnp.int32),
            pltpu.VMEM((128,), jnp.int32),
            pltpu.VMEM((128,), jnp.int32),
            pltpu.VMEM((128,), jnp.int32),
            pltpu.VMEM((3 * _NLOC,), jnp.float32),
            pltpu.VMEM((_NLOC,), jnp.int32),
            pltpu.VMEM((_NLOC,), jnp.float32),
            pltpu.VMEM((_NLOC * 18,), jnp.float32),
            pltpu.SemaphoreType.DMA,
        ],
    )
    return run(boxes_t, s2_t, s3_t, img_flat, fu_tile)


def kernel(img, bboxes_2d, seeds_3d, seeds_2d, calib_K):
    boxes_t = jnp.transpose(bboxes_2d, (0, 2, 1)).reshape(-1)
    s2_t = jnp.transpose(seeds_2d, (0, 2, 1)).reshape(-1)
    s3_t = jnp.transpose(seeds_3d, (0, 2, 1)).reshape(-1)
    fu_tile = jnp.broadcast_to(calib_K[:, 0:1, 0], (_B, 16)).reshape(-1)
    out = _vote_fusion(boxes_t, s2_t, s3_t, img.reshape(-1), fu_tile)
    return out.reshape(_B, _N, 18)


# speedup vs baseline: 1.0844x; 1.0844x over previous
import jax
import jax.numpy as jnp
from jax import lax
from jax.experimental import pallas as pl
from jax.experimental.pallas import tpu as pltpu
from jax.experimental.pallas import tpu_sc as plsc

_B, _K, _N, _H, _W = 2, 128, 4096, 512, 512
_NCLS = 10
_NC, _NS = 2, 16
_NW = _NC * _NS
_NLOC = (_B * _N) // _NW
_NGRP = _NLOC // 16
_GPI = 4
_NINST = _NGRP // _GPI


def _rsqrt(s):
    i = plsc.bitcast(s, jnp.int32)
    i = jnp.int32(0x5F3759DF) - (i >> 1)
    y = plsc.bitcast(i, jnp.float32)
    for _ in range(3):
        y = y * (jnp.float32(1.5) - jnp.float32(0.5) * s * y * y)
    return y


def _splat(v):
    return jnp.full((16,), v, jnp.int32)


def _vote_fusion_body(boxes_hbm, s2_hbm, s3_hbm, img_hbm, fu_hbm, out_hbm,
                      boxes_v, midx_v, midy_v, wk2_v, hk2_v, s2_v, s3_v,
                      fu_v, idx_refs, txt_v, bidx_v, macc_v, out_v, sem, sem_in):
    wid = lax.axis_index("s") * _NC + lax.axis_index("c")
    b = wid // _NS
    base = (wid % _NS) * _NLOC

    in_copies = [
        pltpu.async_copy(boxes_hbm.at[pl.ds(b * 6 * _K, 6 * _K)], boxes_v,
                         sem_in),
        pltpu.async_copy(fu_hbm.at[pl.ds(b * 16, 16)], fu_v, sem_in),
    ]
    for rr in range(2):
        in_copies.append(pltpu.async_copy(
            s2_hbm.at[pl.ds((b * 2 + rr) * _N + base, _NLOC)],
            s2_v.at[pl.ds(rr * _NLOC, _NLOC)], sem_in))
    for rr in range(3):
        in_copies.append(pltpu.async_copy(
            s3_hbm.at[pl.ds((b * 3 + rr) * _N + base, _NLOC)],
            s3_v.at[pl.ds(rr * _NLOC, _NLOC)], sem_in))
    for cp in in_copies:
        cp.wait()

    for i in range(_K // 16):
        sl = pl.ds(i * 16, 16)
        l = boxes_v[pl.ds(0 * _K + i * 16, 16)]
        t = boxes_v[pl.ds(1 * _K + i * 16, 16)]
        r = boxes_v[pl.ds(2 * _K + i * 16, 16)]
        bo = boxes_v[pl.ds(3 * _K + i * 16, 16)]
        midx_v[sl] = (l + r) * 0.5
        midy_v[sl] = (t + bo) * 0.5
        wk2 = (r - l) * 0.5
        hk2 = (bo - t) * 0.5
        wk2_v[sl] = wk2 * wk2
        hk2_v[sl] = hk2 * hk2

    iota = jnp.arange(16, dtype=jnp.int32)
    for g in range(_NGRP):
        xi = s2_v[pl.ds(g * 16, 16)].astype(jnp.int32)
        yi = s2_v[pl.ds(_NLOC + g * 16, 16)].astype(jnp.int32)
        pix = jnp.minimum(jnp.maximum(yi * _W + xi, 0), _H * _W - 1)
        for c in range(3):
            r = c * 2 + g // 8
            idx_refs[r][pl.ds((g % 8) * 16, 16)] = pix + (b * 3 + c) * (_H * _W)
    copies = []
    for c in range(3):
        for j in range(2):
            copies.append(pltpu.async_copy(
                img_hbm.at[idx_refs[c * 2 + j]],
                txt_v.at[pl.ds((c * 2 + j) * 128, 128)], sem))

    for inst in range(_NINST):
        sxs = [s2_v[pl.ds(inst * 64 + j * 16, 16)] for j in range(_GPI)]
        sys_ = [s2_v[pl.ds(_NLOC + inst * 64 + j * 16, 16)]
                for j in range(_GPI)]
        inf = jnp.full((16,), jnp.inf, jnp.float32)
        zero = jnp.zeros((16,), jnp.int32)

        def body(k, carry, sxs=sxs, sys_=sys_):
            best, bidx, macc = carry
            kk = jnp.full((16,), k, jnp.int32)
            mx = plsc.load_gather(midx_v, [kk])
            my = plsc.load_gather(midy_v, [kk])
            wk2s = plsc.load_gather(wk2_v, [kk])
            hk2s = plsc.load_gather(hk2_v, [kk])
            nb, nbi, nm = [], [], []
            for j in range(_GPI):
                du = mx - sxs[j]
                dv = my - sys_[j]
                du2 = du * du
                dv2 = dv * dv
                d2 = du2 + dv2
                upd = d2 < best[j]
                nbi.append(jnp.where(upd, kk, bidx[j]))
                nb.append(jnp.minimum(d2, best[j]))
                m = jnp.maximum(du2 - wk2s, dv2 - hk2s)
                nm.append(jnp.minimum(m, macc[j]))
            return tuple(nb), tuple(nbi), tuple(nm)

        init = ((inf,) * _GPI, (zero,) * _GPI, (inf,) * _GPI)
        _, bidx, macc = lax.fori_loop(0, _K, body, init, unroll=4)
        for j in range(_GPI):
            sl = pl.ds(inst * 64 + j * 16, 16)
            bidx_v[sl] = bidx[j]
            macc_v[sl] = macc[j]

    for cp in copies:
        cp.wait()

    fu = fu_v[...]
    for g in range(_NGRP):
        sx = s2_v[pl.ds(g * 16, 16)]
        sy = s2_v[pl.ds(_NLOC + g * 16, 16)]
        x3 = s3_v[pl.ds(g * 16, 16)]
        y3 = s3_v[pl.ds(_NLOC + g * 16, 16)]
        z3 = s3_v[pl.ds(2 * _NLOC + g * 16, 16)]
        bidx = bidx_v[pl.ds(g * 16, 16)]
        valid = macc_v[pl.ds(g * 16, 16)] < 0.0
        mxa = plsc.load_gather(midx_v, [bidx])
        mya = plsc.load_gather(midy_v, [bidx])
        confa = plsc.load_gather(boxes_v, [bidx + 4 * _K])
        clsa = plsc.load_gather(boxes_v, [bidx + 5 * _K])
        du = mxa - sx
        dv = mya - sy
        zdf = z3 / fu
        g2 = du * zdf + x3
        g3 = dv * zdf + y3
        inv = _rsqrt(g2 * g2 + g3 * g3 + z3 * z3)
        rows = iota + g * 16
        zf = jnp.zeros((16,), jnp.float32)
        for c in range(_NCLS):
            val = jnp.where(valid & (clsa == float(c)), confa, zf)
            plsc.store_scatter(out_v, [rows, _splat(c)], val)
        for c in range(3):
            val = jnp.where(valid, txt_v[pl.ds(c * _NLOC + g * 16, 16)], zf)
            plsc.store_scatter(out_v, [rows, _splat(10 + c)], val)
        geo = (g2, g3, g2 * inv, g3 * inv, z3 * inv)
        for c in range(5):
            val = jnp.where(valid, geo[c], zf)
            plsc.store_scatter(out_v, [rows, _splat(13 + c)], val)

    pltpu.sync_copy(out_v, out_hbm.at[b, pl.ds(base, _NLOC), :])


def _body_wrapper(boxes_hbm, s2_hbm, s3_hbm, img_hbm, fu_hbm, out_hbm,
                  boxes_v, midx_v, midy_v, wk2_v, hk2_v, s2_v, s3_v, fu_v,
                  i0, i1, i2, i3, i4, i5, txt_v, bidx_v, macc_v, out_v, sem,
                  sem_in):
    _vote_fusion_body(boxes_hbm, s2_hbm, s3_hbm, img_hbm, fu_hbm, out_hbm,
                      boxes_v, midx_v, midy_v, wk2_v, hk2_v, s2_v, s3_v,
                      fu_v, [i0, i1, i2, i3, i4, i5], txt_v, bidx_v, macc_v,
                      out_v, sem, sem_in)


@jax.jit
def _vote_fusion(boxes_t, s2_t, s3_t, img_flat, fu_tile):
    run = pl.kernel(
        _body_wrapper,
        out_type=jax.ShapeDtypeStruct((_B, _N, 18), jnp.float32),
        mesh=plsc.VectorSubcoreMesh(
            core_axis_name="c", subcore_axis_name="s",
            num_cores=_NC, num_subcores=_NS),
        compiler_params=pltpu.CompilerParams(needs_layout_passes=False),
        scratch_types=[
            pltpu.VMEM((6 * _K,), jnp.float32),
            pltpu.VMEM((_K,), jnp.float32),
            pltpu.VMEM((_K,), jnp.float32),
            pltpu.VMEM((_K,), jnp.float32),
            pltpu.VMEM((_K,), jnp.float32),
            pltpu.VMEM((2 * _NLOC,), jnp.float32),
            pltpu.VMEM((3 * _NLOC,), jnp.float32),
            pltpu.VMEM((16,), jnp.float32),
            pltpu.VMEM((128,), jnp.int32),
            pltpu.VMEM((128,), jnp.int32),
            pltpu.VMEM((128,), jnp.int32),
            pltpu.VMEM((128,), jnp.int32),
            pltpu.VMEM((128,), jnp.int32),
            pltpu.VMEM((128,), jnp.int32),
            pltpu.VMEM((3 * _NLOC,), jnp.float32),
            pltpu.VMEM((_NLOC,), jnp.int32),
            pltpu.VMEM((_NLOC,), jnp.float32),
            pltpu.VMEM((_NLOC, 18), jnp.float32),
            pltpu.SemaphoreType.DMA,
            pltpu.SemaphoreType.DMA,
        ],
    )
    return run(boxes_t, s2_t, s3_t, img_flat, fu_tile)


def kernel(img, bboxes_2d, seeds_3d, seeds_2d, calib_K):
    boxes_t = jnp.transpose(bboxes_2d, (0, 2, 1)).reshape(-1)
    s2_t = jnp.transpose(seeds_2d, (0, 2, 1)).reshape(-1)
    s3_t = jnp.transpose(seeds_3d, (0, 2, 1)).reshape(-1)
    fu_tile = jnp.broadcast_to(calib_K[:, 0:1, 0], (_B, 16)).reshape(-1)
    return _vote_fusion(boxes_t, s2_t, s3_t, img.reshape(-1), fu_tile)
